# baseline (speedup 1.0000x reference)
import functools

import jax
import jax.numpy as jnp
from jax import lax
from jax.experimental import pallas as pl
from jax.experimental.pallas import tpu as pltpu
from jax.experimental.pallas import tpu_sc as plsc

B = 2
N_IN = 327680
N_OUT = 81920
D = 64

NW = 32
IN_TOT = B * D * N_IN
OUT_TOT = B * D * N_OUT
IN_PER_W = IN_TOT // NW
OUT_PER_W = OUT_TOT // NW
LC = 4096
NCHUNK = OUT_PER_W // LC
NBUF = 4
NPAIR = NCHUNK // NBUF
UNROLL = 32


def _make_kernel():
    mesh = plsc.VectorSubcoreMesh(core_axis_name="c", subcore_axis_name="s")

    @functools.partial(
        pl.kernel,
        mesh=mesh,
        compiler_params=pltpu.CompilerParams(
            use_tc_tiling_on_sc=False, needs_layout_passes=False
        ),
        out_type=jax.ShapeDtypeStruct((OUT_TOT,), jnp.float32),
        scratch_types=(
            [pltpu.VMEM((4 * LC,), jnp.float32) for _ in range(NBUF)]
            + [pltpu.VMEM((LC,), jnp.float32) for _ in range(NBUF)]
            + [pltpu.VMEM((LC,), jnp.int32)]
            + [pltpu.SemaphoreType.DMA for _ in range(3 * NBUF)]
        ),
    )
    def k(x_hbm, out_hbm, in0, in1, in2, in3, ob0, ob1, ob2, ob3, idxb,
          isem0, isem1, isem2, isem3, osem0, osem1, osem2, osem3,
          gsem0, gsem1, gsem2, gsem3):
        ibufs = (in0, in1, in2, in3)
        obufs = (ob0, ob1, ob2, ob3)
        isems = (isem0, isem1, isem2, isem3)
        osems = (osem0, osem1, osem2, osem3)
        gsems = (gsem0, gsem1, gsem2, gsem3)
        wid = lax.axis_index("s") * 2 + lax.axis_index("c")
        base_in = wid * IN_PER_W
        base_out = wid * OUT_PER_W
        iv = 4 * lax.iota(jnp.int32, 16) + 3

        def fill_idx(it, _):
            idxb[pl.ds(it * 16, 16)] = iv + 64 * it
            return 0

        lax.fori_loop(0, LC // 16, fill_idx, 0)

        def in_copy(ci, slot):
            return pltpu.make_async_copy(
                x_hbm.at[pl.ds(base_in + ci * 4 * LC, 4 * LC)],
                ibufs[slot],
                isems[slot],
            )

        def out_copy(ci, slot):
            return pltpu.make_async_copy(
                obufs[slot],
                out_hbm.at[pl.ds(base_out + ci * LC, LC)],
                osems[slot],
            )

        def select(slot):
            src = ibufs[slot]
            dst = obufs[slot]

            def body(it, _):
                k0 = it * (16 * UNROLL)
                for g in range(UNROLL):
                    kk = k0 + g * 16
                    dst[pl.ds(kk, 16)] = plsc.load_gather(src, [iv + 4 * kk])
                return 0

            lax.fori_loop(0, LC // (16 * UNROLL), body, 0)

        for s in range(NBUF):
            in_copy(s, s).start()

        for s in range(NBUF):
            in_copy(s, s).wait()
            select(s)
            out_copy(s, s).start()
            in_copy(s + NBUF, s).start()

        def pair(g, _):
            for s in range(NBUF):
                ci = g * NBUF + s
                in_copy(ci, s).wait()
                out_copy(ci - NBUF, s).wait()
                select(s)
                out_copy(ci, s).start()
                in_copy(ci + NBUF, s).start()
            return 0

        lax.fori_loop(1, NPAIR - 1, pair, 0)

        for s in range(NBUF):
            ci = (NPAIR - 1) * NBUF + s
            in_copy(ci, s).wait()
            out_copy(ci - NBUF, s).wait()
            select(s)
            out_copy(ci, s).start()
        for s in range(NBUF):
            out_copy((NPAIR - 1) * NBUF + s, s).wait()

    return k


_sc_copy = _make_kernel()


@jax.jit
def kernel(x):
    xf = x.transpose(0, 2, 1).reshape(IN_TOT)
    out_f = _sc_copy(xf)
    return out_f.reshape(B, D, N_OUT).transpose(0, 2, 1)


# speedup vs baseline: 1.1611x; 1.1611x over previous
import functools

import jax
import jax.numpy as jnp
from jax import lax
from jax.experimental import pallas as pl
from jax.experimental.pallas import tpu as pltpu
from jax.experimental.pallas import tpu_sc as plsc

B = 2
N_IN = 327680
N_OUT = 81920
D = 64

ROWS = B * N_OUT
NW = 32
ROWS_PER_W = ROWS // NW
CHUNK = 512
NCHUNK = ROWS_PER_W // CHUNK
NBUF = 2


def _make_kernel():
    mesh = plsc.VectorSubcoreMesh(core_axis_name="c", subcore_axis_name="s")

    @functools.partial(
        pl.kernel,
        mesh=mesh,
        out_type=jax.ShapeDtypeStruct((ROWS, D), jnp.float32),
        scratch_types=(
            [pltpu.VMEM((CHUNK, D), jnp.float32) for _ in range(NBUF)]
            + [pltpu.SemaphoreType.DMA for _ in range(2 * NBUF)]
        ),
    )
    def k(x_hbm, out_hbm, buf0, buf1, isem0, isem1, osem0, osem1):
        bufs = (buf0, buf1)
        isems = (isem0, isem1)
        osems = (osem0, osem1)
        wid = lax.axis_index("s") * 2 + lax.axis_index("c")
        base = wid * ROWS_PER_W

        def in_copy(ci, slot):
            off = base + ci * CHUNK
            return pltpu.make_async_copy(
                x_hbm.at[pl.ds(off, CHUNK), 3], bufs[slot], isems[slot]
            )

        def out_copy(ci, slot):
            off = base + ci * CHUNK
            return pltpu.make_async_copy(
                bufs[slot], out_hbm.at[pl.ds(off, CHUNK)], osems[slot]
            )

        for s in range(min(NBUF, NCHUNK)):
            in_copy(s, s).start()

        for ci in range(NCHUNK):
            slot = ci % NBUF
            in_copy(ci, slot).wait()
            out_copy(ci, slot).start()
            out_copy(ci, slot).wait()
            nxt = ci + NBUF
            if nxt < NCHUNK:
                in_copy(nxt, slot).start()

    return k


_sc_copy = _make_kernel()


@jax.jit
def kernel(x):
    xg = x.reshape(ROWS, 4, D)
    out = _sc_copy(xg)
    return out.reshape(B, N_OUT, D)
